# Initial kernel scaffold; baseline (speedup 1.0000x reference)
import jax
import jax.numpy as jnp
from jax.experimental import pallas as pl


def kernel(node_idx, edge_index, batch, emb, Wself0, Wneigh0, b0, Wself1, Wneigh1, b1, Wself2, Wneigh2, b2, gn_gamma0, gn_beta0, gn_alpha0, gn_gamma1, gn_beta1, gn_alpha1, mlp_W1, mlp_b1, mlp_W2, mlp_b2):
    raise NotImplementedError("write your pallas kernel here")


# speedup vs baseline: 1.0525x; 1.0525x over previous
import functools

import jax
import jax.numpy as jnp
from jax.experimental import pallas as pl
from jax.experimental.pallas import tpu as pltpu

N = 10000
E = 320000
D = 128
H = 128
C = 10
G = 64


def _dense_layer_body(x_ref, aggp_ref, deg_ref, ws_ref, wn_ref, b_ref,
                      gamma_ref, beta_ref, alpha_ref, out_ref):
    x = x_ref[...]
    agg = aggp_ref[0] + aggp_ref[1]
    deg = deg_ref[...]
    mean = agg * (1.0 / jnp.maximum(deg, 1.0))
    y = (jnp.dot(x, ws_ref[...], preferred_element_type=jnp.float32)
         + jnp.dot(mean, wn_ref[...], preferred_element_type=jnp.float32)
         + b_ref[...])
    col_mean = jnp.mean(y, axis=0, keepdims=True)
    sub = y - alpha_ref[...] * col_mean
    var = jnp.mean(sub * sub, axis=0, keepdims=True)
    yn = gamma_ref[...] * sub * jax.lax.rsqrt(var + 1e-5) + beta_ref[...]
    out_ref[...] = jnp.maximum(yn, 0.0)


@jax.jit
def _dense_layer(x, aggp, deg, ws, wn, b, gamma, beta, alpha):
    return pl.pallas_call(
        _dense_layer_body,
        out_shape=jax.ShapeDtypeStruct((N, H), jnp.float32),
    )(x, aggp, deg, ws, wn, b.reshape(1, H),
      gamma.reshape(1, H), beta.reshape(1, H), alpha.reshape(1, H))


def _final_body(x_ref, aggp_ref, deg_ref, ws_ref, wn_ref, b_ref,
                batch_ref, w1_ref, b1_ref, w2_ref, b2_ref, out_ref):
    x = x_ref[...]
    agg = aggp_ref[0] + aggp_ref[1]
    deg = deg_ref[...]
    mean = agg * (1.0 / jnp.maximum(deg, 1.0))
    x3 = (jnp.dot(x, ws_ref[...], preferred_element_type=jnp.float32)
          + jnp.dot(mean, wn_ref[...], preferred_element_type=jnp.float32)
          + b_ref[...])
    batch = batch_ref[...]
    gid = jax.lax.broadcasted_iota(jnp.int32, (G, N), 0)
    onehot = (batch == gid).astype(jnp.float32)
    sums = jnp.dot(onehot, x3, preferred_element_type=jnp.float32)
    cnts = jnp.sum(onehot, axis=1, keepdims=True)
    h = sums * (1.0 / jnp.maximum(cnts, 1.0))
    h = jnp.maximum(
        jnp.dot(h, w1_ref[...], preferred_element_type=jnp.float32)
        + b1_ref[...], 0.0)
    out_ref[...] = (jnp.dot(h, w2_ref[...], preferred_element_type=jnp.float32)
                    + b2_ref[...])


@jax.jit
def _final_layer(x, aggp, deg, ws, wn, b, batch, w1, b1, w2, b2):
    return pl.pallas_call(
        _final_body,
        out_shape=jax.ShapeDtypeStruct((G, C), jnp.float32),
    )(x, aggp, deg, ws, wn, b.reshape(1, H), batch.reshape(1, N),
      w1, b1.reshape(1, H), w2, b2.reshape(1, C))


def kernel(node_idx, edge_index, batch, emb, Wself0, Wneigh0, b0, Wself1,
           Wneigh1, b1, Wself2, Wneigh2, b2, gn_gamma0, gn_beta0, gn_alpha0,
           gn_gamma1, gn_beta1, gn_alpha1, mlp_W1, mlp_b1, mlp_W2, mlp_b2):
    src, dst = edge_index[0], edge_index[1]
    x = jnp.take(emb, node_idx, axis=0)
    deg = jax.ops.segment_sum(jnp.ones((E,), jnp.float32), dst,
                              num_segments=N).reshape(N, 1)

    def agg_pair(xc):
        a = jax.ops.segment_sum(xc[src], dst, num_segments=N)
        return jnp.stack([a, jnp.zeros_like(a)])

    x1 = _dense_layer(x, agg_pair(x), deg, Wself0, Wneigh0, b0,
                      gn_gamma0, gn_beta0, gn_alpha0)
    x2 = _dense_layer(x1, agg_pair(x1), deg, Wself1, Wneigh1, b1,
                      gn_gamma1, gn_beta1, gn_alpha1)
    out = _final_layer(x2, agg_pair(x2), deg, Wself2, Wneigh2, b2,
                       batch, mlp_W1, mlp_b1, mlp_W2, mlp_b2)
    return out


# speedup vs baseline: 3.6502x; 3.4683x over previous
import functools

import jax
import jax.numpy as jnp
from jax import lax
from jax.experimental import pallas as pl
from jax.experimental.pallas import tpu as pltpu
from jax.experimental.pallas import tpu_sc as plsc

N = 10000
E = 320000
D = 128
H = 128
C = 10
G = 64

NP = 10240
K = 80
NTILES = 32
EPT = E // NTILES
RPT = NP // 16

_MESH = plsc.VectorSubcoreMesh(core_axis_name="c", subcore_axis_name="s")

_USE_SC_AGG = True


@functools.partial(
    pl.kernel,
    out_type=jax.ShapeDtypeStruct((NP, D), jnp.float32),
    mesh=_MESH,
    scratch_types=[
        pltpu.VMEM((K,), jnp.int32),
        pltpu.VMEM((K, D), jnp.float32),
        pltpu.SemaphoreType.DMA,
    ],
)
def _emb_gather(idx_hbm, emb_hbm, out_hbm, idx_v, rows_v, sem):
    cid = lax.axis_index("c")
    sid = lax.axis_index("s")
    wid = sid * 2 + cid
    base0 = wid * (NP // NTILES)

    def body(j, carry):
        base = base0 + j * K
        pltpu.sync_copy(idx_hbm.at[pl.ds(base, K)], idx_v)
        pltpu.async_copy(emb_hbm.at[idx_v], rows_v, sem).wait()
        pltpu.sync_copy(rows_v, out_hbm.at[pl.ds(base, K)])
        return carry

    lax.fori_loop(0, NP // NTILES // K, body, 0)


_AGG_MODE = 2


def _make_agg(with_deg: bool):
    out_type = [jax.ShapeDtypeStruct((2 * NP, D), jnp.float32)]
    scratch = [
        pltpu.VMEM((K,), jnp.int32),
        pltpu.VMEM((K,), jnp.int32),
        pltpu.VMEM((K, D), jnp.float32),
        pltpu.VMEM_SHARED((NP, D), jnp.float32),
        pltpu.SemaphoreType.DMA,
    ]
    if with_deg:
        out_type.append(jax.ShapeDtypeStruct((2 * NP, 16), jnp.float32))
        scratch += [
            pltpu.VMEM((K, 16), jnp.float32),
            pltpu.VMEM_SHARED((NP, 16), jnp.float32),
        ]

    def body(*refs):
        if with_deg:
            (x_hbm, src_hbm, dst_hbm, z_hbm, z16_hbm, ones_hbm,
             aggp_hbm, degp_hbm,
             src_v, dst_v, rows_v, agg_sh, sem, ones_v, deg_sh) = refs
        else:
            (x_hbm, src_hbm, dst_hbm, z_hbm, aggp_hbm,
             src_v, dst_v, rows_v, agg_sh, sem) = refs
        cid = lax.axis_index("c")
        sid = lax.axis_index("s")

        rbase = sid * RPT
        pltpu.sync_copy(z_hbm.at[pl.ds(rbase, RPT)],
                        agg_sh.at[pl.ds(rbase, RPT)])
        if with_deg:
            pltpu.sync_copy(z16_hbm.at[pl.ds(rbase, RPT)],
                            deg_sh.at[pl.ds(rbase, RPT)])
            pltpu.sync_copy(ones_hbm, ones_v)
        plsc.subcore_barrier()

        ebase = cid * (E // 2) + sid * EPT

        def step(j, carry):
            base = ebase + j * K
            pltpu.sync_copy(src_hbm.at[pl.ds(base, K)], src_v)
            pltpu.sync_copy(dst_hbm.at[pl.ds(base, K)], dst_v)
            if _AGG_MODE >= 1:
                pltpu.async_copy(x_hbm.at[src_v], rows_v, sem).wait()
            if _AGG_MODE >= 2:
                pltpu.sync_copy(rows_v, agg_sh.at[dst_v], add=True)
                if with_deg:
                    pltpu.sync_copy(ones_v, deg_sh.at[dst_v], add=True)
            return carry

        lax.fori_loop(0, EPT // K, step, 0)
        plsc.subcore_barrier()

        obase = cid * NP + rbase
        pltpu.sync_copy(agg_sh.at[pl.ds(rbase, RPT)],
                        aggp_hbm.at[pl.ds(obase, RPT)])
        if with_deg:
            pltpu.sync_copy(deg_sh.at[pl.ds(rbase, RPT)],
                            degp_hbm.at[pl.ds(obase, RPT)])

    return pl.kernel(
        body,
        out_type=tuple(out_type) if with_deg else out_type[0],
        mesh=_MESH,
        scratch_types=scratch,
    )


_agg_with_deg = _make_agg(True)
_agg_plain = _make_agg(False)


def _dense_layer_body(x_ref, aggp_ref, degp_ref, ws_ref, wn_ref, b_ref,
                      gamma_ref, beta_ref, alpha_ref, out_ref):
    x = x_ref[...][:N]
    aggp = aggp_ref[...]
    agg = aggp[:N] + aggp[NP:NP + N]
    degp = degp_ref[...]
    deg = (degp[:N] + degp[NP:NP + N])[:, 0:1]
    mean = agg * (1.0 / jnp.maximum(deg, 1.0))
    y = (jnp.dot(x, ws_ref[...], preferred_element_type=jnp.float32)
         + jnp.dot(mean, wn_ref[...], preferred_element_type=jnp.float32)
         + b_ref[...])
    col_mean = jnp.mean(y, axis=0, keepdims=True)
    sub = y - alpha_ref[...] * col_mean
    var = jnp.mean(sub * sub, axis=0, keepdims=True)
    yn = gamma_ref[...] * sub * jax.lax.rsqrt(var + 1e-5) + beta_ref[...]
    out_ref[...] = jnp.maximum(yn, 0.0)


def _dense_layer(x, aggp, degp, ws, wn, b, gamma, beta, alpha):
    return pl.pallas_call(
        _dense_layer_body,
        out_shape=jax.ShapeDtypeStruct((N, H), jnp.float32),
    )(x, aggp, degp, ws, wn, b.reshape(1, H),
      gamma.reshape(1, H), beta.reshape(1, H), alpha.reshape(1, H))


def _final_body(x_ref, aggp_ref, degp_ref, ws_ref, wn_ref, b_ref,
                batch_ref, w1_ref, b1_ref, w2_ref, b2_ref, out_ref):
    x = x_ref[...][:N]
    aggp = aggp_ref[...]
    agg = aggp[:N] + aggp[NP:NP + N]
    degp = degp_ref[...]
    deg = (degp[:N] + degp[NP:NP + N])[:, 0:1]
    mean = agg * (1.0 / jnp.maximum(deg, 1.0))
    x3 = (jnp.dot(x, ws_ref[...], preferred_element_type=jnp.float32)
          + jnp.dot(mean, wn_ref[...], preferred_element_type=jnp.float32)
          + b_ref[...])
    batch = batch_ref[...]
    gid = jax.lax.broadcasted_iota(jnp.int32, (G, N), 0)
    onehot = (batch == gid).astype(jnp.float32)
    sums = jnp.dot(onehot, x3, preferred_element_type=jnp.float32)
    cnts = jnp.sum(onehot, axis=1, keepdims=True)
    h = sums * (1.0 / jnp.maximum(cnts, 1.0))
    h = jnp.maximum(
        jnp.dot(h, w1_ref[...], preferred_element_type=jnp.float32)
        + b1_ref[...], 0.0)
    out_ref[...] = (jnp.dot(h, w2_ref[...], preferred_element_type=jnp.float32)
                    + b2_ref[...])


def _final_layer(x, aggp, degp, ws, wn, b, batch, w1, b1, w2, b2):
    return pl.pallas_call(
        _final_body,
        out_shape=jax.ShapeDtypeStruct((G, C), jnp.float32),
    )(x, aggp, degp, ws, wn, b.reshape(1, H), batch.reshape(1, N),
      w1, b1.reshape(1, H), w2, b2.reshape(1, C))


def kernel(node_idx, edge_index, batch, emb, Wself0, Wneigh0, b0, Wself1,
           Wneigh1, b1, Wself2, Wneigh2, b2, gn_gamma0, gn_beta0, gn_alpha0,
           gn_gamma1, gn_beta1, gn_alpha1, mlp_W1, mlp_b1, mlp_W2, mlp_b2):
    node_idx = node_idx.astype(jnp.int32)
    src = edge_index[0].astype(jnp.int32)
    dst = edge_index[1].astype(jnp.int32)
    batch = batch.astype(jnp.int32)

    idx_pad = jnp.concatenate(
        [node_idx, jnp.zeros((NP - N,), jnp.int32)])

    x0p = _emb_gather(idx_pad, emb)

    zeros_nd = jnp.zeros((NP, D), jnp.float32)
    zeros_16 = jnp.zeros((NP, 16), jnp.float32)
    ones_k16 = jnp.ones((K, 16), jnp.float32)
    if _USE_SC_AGG and _AGG_MODE >= 2:
        dg = jax.ops.segment_sum(jnp.ones((E,), jnp.float32), dst,
                                 num_segments=N)
        degp = jnp.concatenate(
            [jnp.pad(jnp.broadcast_to(dg[:, None], (N, 16)),
                     ((0, NP - N), (0, 0))),
             jnp.zeros((NP, 16), jnp.float32)])
        aggp0 = _agg_plain(x0p, src, dst, zeros_nd)
        aggp1_fn = lambda x: _agg_plain(x, src, dst, zeros_nd)
    else:
        def _jax_aggp(x):
            a = jax.ops.segment_sum(x[src], dst, num_segments=N)
            return jnp.concatenate(
                [jnp.pad(a, ((0, NP - N), (0, 0))),
                 jnp.zeros((NP, D), jnp.float32)])
        dg = jax.ops.segment_sum(jnp.ones((E,), jnp.float32), dst,
                                 num_segments=N)
        degp = jnp.concatenate(
            [jnp.pad(jnp.broadcast_to(dg[:, None], (N, 16)),
                     ((0, NP - N), (0, 0))),
             jnp.zeros((NP, 16), jnp.float32)])
        aggp0 = _jax_aggp(x0p[:N])
        aggp1_fn = _jax_aggp
        if _USE_SC_AGG:
            sc_a = _agg_plain(x0p, src, dst, zeros_nd)
            aggp0 = aggp0 + sc_a * 0.0

    x1 = _dense_layer(x0p, aggp0, degp, Wself0, Wneigh0, b0,
                      gn_gamma0, gn_beta0, gn_alpha0)
    aggp1 = aggp1_fn(x1)
    x2 = _dense_layer(x1, aggp1, degp, Wself1, Wneigh1, b1,
                      gn_gamma1, gn_beta1, gn_alpha1)
    aggp2 = aggp1_fn(x2)
    return _final_layer(x2, aggp2, degp, Wself2, Wneigh2, b2,
                        batch, mlp_W1, mlp_b1, mlp_W2, mlp_b2)


# speedup vs baseline: 4.4138x; 1.2092x over previous
import functools

import jax
import jax.numpy as jnp
from jax import lax
from jax.experimental import pallas as pl
from jax.experimental.pallas import tpu as pltpu
from jax.experimental.pallas import tpu_sc as plsc

N = 10000
E = 320000
D = 128
H = 128
C = 10
G = 64

NP = 10240
K = 80
NTILES = 32
EPT = E // NTILES
RPT = NP // 16

_MESH = plsc.VectorSubcoreMesh(core_axis_name="c", subcore_axis_name="s")


@functools.partial(
    pl.kernel,
    out_type=jax.ShapeDtypeStruct((NP, D), jnp.float32),
    mesh=_MESH,
    scratch_types=[
        pltpu.VMEM((K,), jnp.int32),
        pltpu.VMEM((K, D), jnp.float32),
        pltpu.SemaphoreType.DMA,
    ],
)
def _emb_gather(idx_hbm, emb_hbm, out_hbm, idx_v, rows_v, sem):
    cid = lax.axis_index("c")
    sid = lax.axis_index("s")
    wid = sid * 2 + cid
    base0 = wid * (NP // NTILES)

    def body(j, carry):
        base = base0 + j * K
        pltpu.sync_copy(idx_hbm.at[pl.ds(base, K)], idx_v)
        pltpu.async_copy(emb_hbm.at[idx_v], rows_v, sem).wait()
        pltpu.sync_copy(rows_v, out_hbm.at[pl.ds(base, K)])
        return carry

    lax.fori_loop(0, NP // NTILES // K, body, 0)


@functools.partial(
    pl.kernel,
    out_type=jax.ShapeDtypeStruct((2 * NP, D), jnp.float32),
    mesh=_MESH,
    scratch_types=[
        pltpu.VMEM((K,), jnp.int32),
        pltpu.VMEM((K,), jnp.int32),
        pltpu.VMEM((K, D), jnp.float32),
        pltpu.VMEM_SHARED((NP, D), jnp.float32),
        pltpu.SemaphoreType.DMA,
    ],
)
def _agg_plain(x_hbm, src_hbm, dst_hbm, z_hbm, aggp_hbm,
               src_v, dst_v, rows_v, agg_sh, sem):
    cid = lax.axis_index("c")
    sid = lax.axis_index("s")

    rbase = sid * RPT
    pltpu.sync_copy(z_hbm.at[pl.ds(rbase, RPT)],
                    agg_sh.at[pl.ds(rbase, RPT)])
    plsc.subcore_barrier()

    ebase = cid * (E // 2) + sid * EPT

    def step(j, carry):
        base = ebase + j * K
        pltpu.sync_copy(src_hbm.at[pl.ds(base, K)], src_v)
        pltpu.sync_copy(dst_hbm.at[pl.ds(base, K)], dst_v)
        pltpu.async_copy(x_hbm.at[src_v], rows_v, sem).wait()
        pltpu.sync_copy(rows_v, agg_sh.at[dst_v], add=True)
        return carry

    lax.fori_loop(0, EPT // K, step, 0)
    plsc.subcore_barrier()

    obase = cid * NP + rbase
    pltpu.sync_copy(agg_sh.at[pl.ds(rbase, RPT)],
                    aggp_hbm.at[pl.ds(obase, RPT)])


@functools.partial(
    pl.kernel,
    out_type=jax.ShapeDtypeStruct((2 * NP, D), jnp.float32),
    mesh=_MESH,
    scratch_types=[
        pltpu.VMEM((K,), jnp.int32),
        pltpu.VMEM((K, D), jnp.float32),
        pltpu.VMEM_SHARED((NP, D), jnp.float32),
    ],
)
def _deg_scatter(dst_hbm, z_hbm, ones_hbm, degp_hbm, dst_v, ones_v, deg_sh):
    cid = lax.axis_index("c")
    sid = lax.axis_index("s")

    rbase = sid * RPT
    pltpu.sync_copy(z_hbm.at[pl.ds(rbase, RPT)],
                    deg_sh.at[pl.ds(rbase, RPT)])
    pltpu.sync_copy(ones_hbm, ones_v)
    plsc.subcore_barrier()

    ebase = cid * (E // 2) + sid * EPT

    def step(j, carry):
        base = ebase + j * K
        pltpu.sync_copy(dst_hbm.at[pl.ds(base, K)], dst_v)
        pltpu.sync_copy(ones_v, deg_sh.at[dst_v], add=True)
        return carry

    lax.fori_loop(0, EPT // K, step, 0)
    plsc.subcore_barrier()

    obase = cid * NP + rbase
    pltpu.sync_copy(deg_sh.at[pl.ds(rbase, RPT)],
                    degp_hbm.at[pl.ds(obase, RPT)])


def _dense_layer_body(x_ref, aggp_ref, degp_ref, ws_ref, wn_ref, b_ref,
                      gamma_ref, beta_ref, alpha_ref, out_ref):
    x = x_ref[...][:N]
    aggp = aggp_ref[...]
    agg = aggp[:N] + aggp[NP:NP + N]
    deg = degp_ref[...]
    mean = agg * (1.0 / jnp.maximum(deg, 1.0))
    y = (jnp.dot(x, ws_ref[...], preferred_element_type=jnp.float32)
         + jnp.dot(mean, wn_ref[...], preferred_element_type=jnp.float32)
         + b_ref[...])
    col_mean = jnp.mean(y, axis=0, keepdims=True)
    sub = y - alpha_ref[...] * col_mean
    var = jnp.mean(sub * sub, axis=0, keepdims=True)
    yn = gamma_ref[...] * sub * jax.lax.rsqrt(var + 1e-5) + beta_ref[...]
    out_ref[...] = jnp.maximum(yn, 0.0)


def _dense_layer(x, aggp, degp, ws, wn, b, gamma, beta, alpha):
    return pl.pallas_call(
        _dense_layer_body,
        out_shape=jax.ShapeDtypeStruct((N, H), jnp.float32),
    )(x, aggp, degp, ws, wn, b.reshape(1, H),
      gamma.reshape(1, H), beta.reshape(1, H), alpha.reshape(1, H))


def _final_body(x_ref, aggp_ref, degp_ref, ws_ref, wn_ref, b_ref,
                batch_ref, w1_ref, b1_ref, w2_ref, b2_ref, out_ref):
    x = x_ref[...][:N]
    aggp = aggp_ref[...]
    agg = aggp[:N] + aggp[NP:NP + N]
    deg = degp_ref[...]
    mean = agg * (1.0 / jnp.maximum(deg, 1.0))
    x3 = (jnp.dot(x, ws_ref[...], preferred_element_type=jnp.float32)
          + jnp.dot(mean, wn_ref[...], preferred_element_type=jnp.float32)
          + b_ref[...])
    batch = batch_ref[...]
    gid = jax.lax.broadcasted_iota(jnp.int32, (G, N), 0)
    onehot = (batch == gid).astype(jnp.float32)
    sums = jnp.dot(onehot, x3, preferred_element_type=jnp.float32)
    cnts = jnp.sum(onehot, axis=1, keepdims=True)
    h = sums * (1.0 / jnp.maximum(cnts, 1.0))
    h = jnp.maximum(
        jnp.dot(h, w1_ref[...], preferred_element_type=jnp.float32)
        + b1_ref[...], 0.0)
    out_ref[...] = (jnp.dot(h, w2_ref[...], preferred_element_type=jnp.float32)
                    + b2_ref[...])


def _final_layer(x, aggp, degp, ws, wn, b, batch, w1, b1, w2, b2):
    return pl.pallas_call(
        _final_body,
        out_shape=jax.ShapeDtypeStruct((G, C), jnp.float32),
    )(x, aggp, degp, ws, wn, b.reshape(1, H), batch.reshape(1, N),
      w1, b1.reshape(1, H), w2, b2.reshape(1, C))


def kernel(node_idx, edge_index, batch, emb, Wself0, Wneigh0, b0, Wself1,
           Wneigh1, b1, Wself2, Wneigh2, b2, gn_gamma0, gn_beta0, gn_alpha0,
           gn_gamma1, gn_beta1, gn_alpha1, mlp_W1, mlp_b1, mlp_W2, mlp_b2):
    node_idx = node_idx.astype(jnp.int32)
    src = edge_index[0].astype(jnp.int32)
    dst = edge_index[1].astype(jnp.int32)
    batch = batch.astype(jnp.int32)

    idx_pad = jnp.concatenate(
        [node_idx, jnp.zeros((NP - N,), jnp.int32)])

    x0p = _emb_gather(idx_pad, emb)

    zeros_nd = jnp.zeros((NP, D), jnp.float32)
    ones_kd = jnp.ones((K, D), jnp.float32)
    degp2 = _deg_scatter(dst, zeros_nd, ones_kd)
    degp = degp2[:N, :1] + degp2[NP:NP + N, :1]
    aggp0 = _agg_plain(x0p, src, dst, zeros_nd)
    aggp1_fn = lambda x: _agg_plain(x, src, dst, zeros_nd)

    x1 = _dense_layer(x0p, aggp0, degp, Wself0, Wneigh0, b0,
                      gn_gamma0, gn_beta0, gn_alpha0)
    aggp1 = aggp1_fn(x1)
    x2 = _dense_layer(x1, aggp1, degp, Wself1, Wneigh1, b1,
                      gn_gamma1, gn_beta1, gn_alpha1)
    aggp2 = aggp1_fn(x2)
    return _final_layer(x2, aggp2, degp, Wself2, Wneigh2, b2,
                        batch, mlp_W1, mlp_b1, mlp_W2, mlp_b2)


# speedup vs baseline: 5.3811x; 1.2191x over previous
import functools

import jax
import jax.numpy as jnp
from jax import lax
from jax.experimental import pallas as pl
from jax.experimental.pallas import tpu as pltpu
from jax.experimental.pallas import tpu_sc as plsc

N = 10000
E = 320000
D = 128
H = 128
C = 10
G = 64

NP = 10240
K = 80
NTILES = 32
EPT = E // NTILES
RPT = NP // 16

_MESH = plsc.VectorSubcoreMesh(core_axis_name="c", subcore_axis_name="s")


@functools.partial(
    pl.kernel,
    out_type=jax.ShapeDtypeStruct((NP, D), jnp.float32),
    mesh=_MESH,
    scratch_types=[
        pltpu.VMEM((K,), jnp.int32),
        pltpu.VMEM((K, D), jnp.float32),
        pltpu.SemaphoreType.DMA,
    ],
)
def _emb_gather(idx_hbm, emb_hbm, out_hbm, idx_v, rows_v, sem):
    cid = lax.axis_index("c")
    sid = lax.axis_index("s")
    wid = sid * 2 + cid
    base0 = wid * (NP // NTILES)

    def body(j, carry):
        base = base0 + j * K
        pltpu.sync_copy(idx_hbm.at[pl.ds(base, K)], idx_v)
        pltpu.async_copy(emb_hbm.at[idx_v], rows_v, sem).wait()
        pltpu.sync_copy(rows_v, out_hbm.at[pl.ds(base, K)])
        return carry

    lax.fori_loop(0, NP // NTILES // K, body, 0)


@functools.partial(
    pl.kernel,
    out_type=jax.ShapeDtypeStruct((2 * NP, D), jnp.float32),
    mesh=_MESH,
    scratch_types=[
        pltpu.VMEM((K,), jnp.int32),
        pltpu.VMEM((K,), jnp.int32),
        pltpu.VMEM((K,), jnp.int32),
        pltpu.VMEM((K,), jnp.int32),
        pltpu.VMEM((K, D), jnp.float32),
        pltpu.VMEM((K, D), jnp.float32),
        pltpu.VMEM_SHARED((NP, D), jnp.float32),
        pltpu.SemaphoreType.DMA,
        pltpu.SemaphoreType.DMA,
        pltpu.SemaphoreType.DMA,
        pltpu.SemaphoreType.DMA,
    ],
)
def _agg_plain(x_hbm, src_hbm, dst_hbm, z_hbm, aggp_hbm,
               src_v0, src_v1, dst_v0, dst_v1, rows_v0, rows_v1, agg_sh,
               gsem0, gsem1, ssem0, ssem1):
    cid = lax.axis_index("c")
    sid = lax.axis_index("s")

    rbase = sid * RPT
    pltpu.sync_copy(z_hbm.at[pl.ds(rbase, RPT)],
                    agg_sh.at[pl.ds(rbase, RPT)])
    plsc.subcore_barrier()

    ebase = cid * (E // 2) + sid * EPT
    srcs = (src_v0, src_v1)
    dsts = (dst_v0, dst_v1)
    rows = (rows_v0, rows_v1)
    gsems = (gsem0, gsem1)
    ssems = (ssem0, ssem1)

    def issue_gather(base, b):
        pltpu.sync_copy(src_hbm.at[pl.ds(base, K)], srcs[b])
        pltpu.sync_copy(dst_hbm.at[pl.ds(base, K)], dsts[b])
        pltpu.async_copy(x_hbm.at[srcs[b]], rows[b], gsems[b])

    def wait_gather(b):
        pltpu.make_async_copy(x_hbm.at[srcs[b]], rows[b], gsems[b]).wait()

    def issue_scatter(b):
        pltpu.async_copy(rows[b], agg_sh.at[dsts[b]], ssems[b], add=True)

    def wait_scatter(b):
        pltpu.make_async_copy(rows[b], agg_sh.at[dsts[b]],
                              ssems[b]).wait()

    nchunk = EPT // K
    issue_gather(ebase, 0)

    def pair(m, carry):
        j = 2 * m
        wait_gather(0)
        issue_scatter(0)
        issue_gather(ebase + (j + 1) * K, 1)
        wait_scatter(0)
        wait_gather(1)
        issue_scatter(1)
        issue_gather(ebase + (j + 2) * K, 0)
        wait_scatter(1)
        return carry

    lax.fori_loop(0, (nchunk - 1) // 2, pair, 0)
    wait_gather(0)
    issue_scatter(0)
    wait_scatter(0)
    plsc.subcore_barrier()

    obase = cid * NP + rbase
    pltpu.sync_copy(agg_sh.at[pl.ds(rbase, RPT)],
                    aggp_hbm.at[pl.ds(obase, RPT)])


@functools.partial(
    pl.kernel,
    out_type=jax.ShapeDtypeStruct((2 * NP, D), jnp.float32),
    mesh=_MESH,
    scratch_types=[
        pltpu.VMEM((K,), jnp.int32),
        pltpu.VMEM((K,), jnp.int32),
        pltpu.VMEM((K, D), jnp.float32),
        pltpu.VMEM_SHARED((NP, D), jnp.float32),
        pltpu.SemaphoreType.DMA,
        pltpu.SemaphoreType.DMA,
    ],
)
def _deg_scatter(dst_hbm, z_hbm, ones_hbm, degp_hbm, dst_v0, dst_v1,
                 ones_v, deg_sh, ssem0, ssem1):
    cid = lax.axis_index("c")
    sid = lax.axis_index("s")

    rbase = sid * RPT
    pltpu.sync_copy(z_hbm.at[pl.ds(rbase, RPT)],
                    deg_sh.at[pl.ds(rbase, RPT)])
    pltpu.sync_copy(ones_hbm, ones_v)
    plsc.subcore_barrier()

    ebase = cid * (E // 2) + sid * EPT
    dsts = (dst_v0, dst_v1)
    ssems = (ssem0, ssem1)

    def issue(b):
        pltpu.async_copy(ones_v, deg_sh.at[dsts[b]], ssems[b], add=True)

    def wait(b):
        pltpu.make_async_copy(ones_v, deg_sh.at[dsts[b]], ssems[b]).wait()

    pltpu.sync_copy(dst_hbm.at[pl.ds(ebase, K)], dst_v0)

    def pair(m, carry):
        j = 2 * m
        issue(0)
        pltpu.sync_copy(dst_hbm.at[pl.ds(ebase + (j + 1) * K, K)], dst_v1)
        wait(0)
        issue(1)
        pltpu.sync_copy(dst_hbm.at[pl.ds(ebase + (j + 2) * K, K)], dst_v0)
        wait(1)
        return carry

    lax.fori_loop(0, (EPT // K - 1) // 2, pair, 0)
    issue(0)
    wait(0)
    plsc.subcore_barrier()

    obase = cid * NP + rbase
    pltpu.sync_copy(deg_sh.at[pl.ds(rbase, RPT)],
                    degp_hbm.at[pl.ds(obase, RPT)])


def _dense_layer_body(x_ref, aggp_ref, degp_ref, ws_ref, wn_ref, b_ref,
                      gamma_ref, beta_ref, alpha_ref, out_ref):
    x = x_ref[...][:N]
    aggp = aggp_ref[...]
    agg = aggp[:N] + aggp[NP:NP + N]
    deg = degp_ref[...]
    mean = agg * (1.0 / jnp.maximum(deg, 1.0))
    y = (jnp.dot(x, ws_ref[...], preferred_element_type=jnp.float32)
         + jnp.dot(mean, wn_ref[...], preferred_element_type=jnp.float32)
         + b_ref[...])
    col_mean = jnp.mean(y, axis=0, keepdims=True)
    sub = y - alpha_ref[...] * col_mean
    var = jnp.mean(sub * sub, axis=0, keepdims=True)
    yn = gamma_ref[...] * sub * jax.lax.rsqrt(var + 1e-5) + beta_ref[...]
    out_ref[...] = jnp.maximum(yn, 0.0)


def _dense_layer(x, aggp, degp, ws, wn, b, gamma, beta, alpha):
    return pl.pallas_call(
        _dense_layer_body,
        out_shape=jax.ShapeDtypeStruct((N, H), jnp.float32),
    )(x, aggp, degp, ws, wn, b.reshape(1, H),
      gamma.reshape(1, H), beta.reshape(1, H), alpha.reshape(1, H))


def _final_body(x_ref, aggp_ref, degp_ref, ws_ref, wn_ref, b_ref,
                batch_ref, w1_ref, b1_ref, w2_ref, b2_ref, out_ref):
    x = x_ref[...][:N]
    aggp = aggp_ref[...]
    agg = aggp[:N] + aggp[NP:NP + N]
    deg = degp_ref[...]
    mean = agg * (1.0 / jnp.maximum(deg, 1.0))
    x3 = (jnp.dot(x, ws_ref[...], preferred_element_type=jnp.float32)
          + jnp.dot(mean, wn_ref[...], preferred_element_type=jnp.float32)
          + b_ref[...])
    batch = batch_ref[...]
    gid = jax.lax.broadcasted_iota(jnp.int32, (G, N), 0)
    onehot = (batch == gid).astype(jnp.float32)
    sums = jnp.dot(onehot, x3, preferred_element_type=jnp.float32)
    cnts = jnp.sum(onehot, axis=1, keepdims=True)
    h = sums * (1.0 / jnp.maximum(cnts, 1.0))
    h = jnp.maximum(
        jnp.dot(h, w1_ref[...], preferred_element_type=jnp.float32)
        + b1_ref[...], 0.0)
    out_ref[...] = (jnp.dot(h, w2_ref[...], preferred_element_type=jnp.float32)
                    + b2_ref[...])


def _final_layer(x, aggp, degp, ws, wn, b, batch, w1, b1, w2, b2):
    return pl.pallas_call(
        _final_body,
        out_shape=jax.ShapeDtypeStruct((G, C), jnp.float32),
    )(x, aggp, degp, ws, wn, b.reshape(1, H), batch.reshape(1, N),
      w1, b1.reshape(1, H), w2, b2.reshape(1, C))


def kernel(node_idx, edge_index, batch, emb, Wself0, Wneigh0, b0, Wself1,
           Wneigh1, b1, Wself2, Wneigh2, b2, gn_gamma0, gn_beta0, gn_alpha0,
           gn_gamma1, gn_beta1, gn_alpha1, mlp_W1, mlp_b1, mlp_W2, mlp_b2):
    node_idx = node_idx.astype(jnp.int32)
    src = edge_index[0].astype(jnp.int32)
    dst = edge_index[1].astype(jnp.int32)
    batch = batch.astype(jnp.int32)

    idx_pad = jnp.concatenate(
        [node_idx, jnp.zeros((NP - N,), jnp.int32)])

    x0p = _emb_gather(idx_pad, emb)

    zeros_nd = jnp.zeros((NP, D), jnp.float32)
    ones_kd = jnp.ones((K, D), jnp.float32)
    degp2 = _deg_scatter(dst, zeros_nd, ones_kd)
    degp = degp2[:N, :1] + degp2[NP:NP + N, :1]
    aggp0 = _agg_plain(x0p, src, dst, zeros_nd)
    aggp1_fn = lambda x: _agg_plain(x, src, dst, zeros_nd)

    x1 = _dense_layer(x0p, aggp0, degp, Wself0, Wneigh0, b0,
                      gn_gamma0, gn_beta0, gn_alpha0)
    aggp1 = aggp1_fn(x1)
    x2 = _dense_layer(x1, aggp1, degp, Wself1, Wneigh1, b1,
                      gn_gamma1, gn_beta1, gn_alpha1)
    aggp2 = aggp1_fn(x2)
    return _final_layer(x2, aggp2, degp, Wself2, Wneigh2, b2,
                        batch, mlp_W1, mlp_b1, mlp_W2, mlp_b2)


# speedup vs baseline: 8.4590x; 1.5720x over previous
import functools

import jax
import jax.numpy as jnp
from jax import lax
from jax.experimental import pallas as pl
from jax.experimental.pallas import tpu as pltpu
from jax.experimental.pallas import tpu_sc as plsc

N = 10000
E = 320000
D = 128
H = 128
C = 10
G = 64

NP = 10240
K = 80
NTILES = 32
EPT = E // NTILES
RPT = NP // 16

_MESH = plsc.VectorSubcoreMesh(core_axis_name="c", subcore_axis_name="s")


@functools.partial(
    pl.kernel,
    out_type=jax.ShapeDtypeStruct((NP, D), jnp.float32),
    mesh=_MESH,
    scratch_types=[
        pltpu.VMEM((K,), jnp.int32),
        pltpu.VMEM((K, D), jnp.float32),
        pltpu.SemaphoreType.DMA,
    ],
)
def _emb_gather(idx_hbm, emb_hbm, out_hbm, idx_v, rows_v, sem):
    cid = lax.axis_index("c")
    sid = lax.axis_index("s")
    wid = sid * 2 + cid
    base0 = wid * (NP // NTILES)

    def body(j, carry):
        base = base0 + j * K
        pltpu.sync_copy(idx_hbm.at[pl.ds(base, K)], idx_v)
        pltpu.async_copy(emb_hbm.at[idx_v], rows_v, sem).wait()
        pltpu.sync_copy(rows_v, out_hbm.at[pl.ds(base, K)])
        return carry

    lax.fori_loop(0, NP // NTILES // K, body, 0)


NCHUNK = EPT // K


@functools.partial(
    pl.kernel,
    out_type=jax.ShapeDtypeStruct((2 * NP, D), jnp.float32),
    mesh=_MESH,
    scratch_types=[
        pltpu.VMEM((EPT,), jnp.int32),
        pltpu.VMEM((K,), jnp.int32),
        pltpu.VMEM((K,), jnp.int32),
        pltpu.VMEM((K, D), jnp.float32),
        pltpu.VMEM((K, D), jnp.float32),
        pltpu.VMEM_SHARED((NP, D), jnp.float32),
        pltpu.SemaphoreType.DMA,
        pltpu.SemaphoreType.DMA,
        pltpu.SemaphoreType.DMA,
        pltpu.SemaphoreType.DMA,
    ],
)
def _agg_plain(x_hbm, src_hbm, dst_hbm, z_hbm, aggp_hbm,
               src_all, dst_v0, dst_v1, rows_v0, rows_v1, agg_sh,
               gsem0, gsem1, ssem0, ssem1):
    cid = lax.axis_index("c")
    sid = lax.axis_index("s")

    rbase = sid * RPT
    pltpu.sync_copy(z_hbm.at[pl.ds(rbase, RPT)],
                    agg_sh.at[pl.ds(rbase, RPT)])
    ebase = cid * (E // 2) + sid * EPT
    pltpu.sync_copy(src_hbm.at[pl.ds(ebase, EPT)], src_all)
    plsc.subcore_barrier()

    dsts = (dst_v0, dst_v1)
    rows = (rows_v0, rows_v1)
    gsems = (gsem0, gsem1)
    ssems = (ssem0, ssem1)

    def copy_dst(j, b):
        pltpu.sync_copy(dst_hbm.at[pl.ds(ebase + j * K, K)], dsts[b])

    def issue_gather(j, b):
        pltpu.async_copy(x_hbm.at[src_all.at[pl.ds(j * K, K)]], rows[b],
                         gsems[b])

    def wait_gather(b):
        pltpu.make_async_copy(x_hbm.at[src_all.at[pl.ds(0, K)]], rows[b],
                              gsems[b]).wait()

    def issue_scatter(b):
        pltpu.async_copy(rows[b], agg_sh.at[dsts[b]], ssems[b], add=True)

    def wait_scatter(b):
        pltpu.make_async_copy(rows[b], agg_sh.at[dsts[b]], ssems[b]).wait()

    copy_dst(0, 0)
    copy_dst(1, 1)
    issue_gather(0, 0)
    issue_gather(1, 1)

    def pair(m, carry):
        j = 2 * m
        wait_gather(0)
        issue_scatter(0)
        wait_scatter(0)
        copy_dst(j + 2, 0)
        issue_gather(j + 2, 0)
        wait_gather(1)
        issue_scatter(1)
        wait_scatter(1)
        nxt = jnp.minimum(j + 3, NCHUNK - 1)
        copy_dst(nxt, 1)
        issue_gather(nxt, 1)
        return carry

    lax.fori_loop(0, (NCHUNK - 1) // 2, pair, 0)
    wait_gather(0)
    issue_scatter(0)
    wait_scatter(0)
    wait_gather(1)
    plsc.subcore_barrier()

    obase = cid * NP + rbase
    pltpu.sync_copy(agg_sh.at[pl.ds(rbase, RPT)],
                    aggp_hbm.at[pl.ds(obase, RPT)])


@functools.partial(
    pl.kernel,
    out_type=jax.ShapeDtypeStruct((2 * NP, D), jnp.float32),
    mesh=_MESH,
    scratch_types=[
        pltpu.VMEM((K,), jnp.int32),
        pltpu.VMEM((K,), jnp.int32),
        pltpu.VMEM((K, D), jnp.float32),
        pltpu.VMEM_SHARED((NP, D), jnp.float32),
        pltpu.SemaphoreType.DMA,
        pltpu.SemaphoreType.DMA,
    ],
)
def _deg_scatter(dst_hbm, z_hbm, ones_hbm, degp_hbm, dst_v0, dst_v1,
                 ones_v, deg_sh, ssem0, ssem1):
    cid = lax.axis_index("c")
    sid = lax.axis_index("s")

    rbase = sid * RPT
    pltpu.sync_copy(z_hbm.at[pl.ds(rbase, RPT)],
                    deg_sh.at[pl.ds(rbase, RPT)])
    pltpu.sync_copy(ones_hbm, ones_v)
    plsc.subcore_barrier()

    ebase = cid * (E // 2) + sid * EPT
    dsts = (dst_v0, dst_v1)
    ssems = (ssem0, ssem1)

    def issue(b):
        pltpu.async_copy(ones_v, deg_sh.at[dsts[b]], ssems[b], add=True)

    def wait(b):
        pltpu.make_async_copy(ones_v, deg_sh.at[dsts[b]], ssems[b]).wait()

    pltpu.sync_copy(dst_hbm.at[pl.ds(ebase, K)], dst_v0)

    def pair(m, carry):
        j = 2 * m
        issue(0)
        pltpu.sync_copy(dst_hbm.at[pl.ds(ebase + (j + 1) * K, K)], dst_v1)
        wait(0)
        issue(1)
        pltpu.sync_copy(dst_hbm.at[pl.ds(ebase + (j + 2) * K, K)], dst_v0)
        wait(1)
        return carry

    lax.fori_loop(0, (EPT // K - 1) // 2, pair, 0)
    issue(0)
    wait(0)
    plsc.subcore_barrier()

    obase = cid * NP + rbase
    pltpu.sync_copy(deg_sh.at[pl.ds(rbase, RPT)],
                    degp_hbm.at[pl.ds(obase, RPT)])


def _dense_layer_body(x_ref, aggp_ref, degp_ref, ws_ref, wn_ref, b_ref,
                      gamma_ref, beta_ref, alpha_ref, out_ref):
    x = x_ref[...][:N]
    aggp = aggp_ref[...]
    agg = aggp[:N] + aggp[NP:NP + N]
    deg = degp_ref[...]
    mean = agg * (1.0 / jnp.maximum(deg, 1.0))
    y = (jnp.dot(x, ws_ref[...], preferred_element_type=jnp.float32)
         + jnp.dot(mean, wn_ref[...], preferred_element_type=jnp.float32)
         + b_ref[...])
    col_mean = jnp.mean(y, axis=0, keepdims=True)
    sub = y - alpha_ref[...] * col_mean
    var = jnp.mean(sub * sub, axis=0, keepdims=True)
    yn = gamma_ref[...] * sub * jax.lax.rsqrt(var + 1e-5) + beta_ref[...]
    out_ref[...] = jnp.maximum(yn, 0.0)


def _dense_layer(x, aggp, degp, ws, wn, b, gamma, beta, alpha):
    return pl.pallas_call(
        _dense_layer_body,
        out_shape=jax.ShapeDtypeStruct((N, H), jnp.float32),
    )(x, aggp, degp, ws, wn, b.reshape(1, H),
      gamma.reshape(1, H), beta.reshape(1, H), alpha.reshape(1, H))


def _final_body(x_ref, aggp_ref, degp_ref, ws_ref, wn_ref, b_ref,
                batch_ref, w1_ref, b1_ref, w2_ref, b2_ref, out_ref):
    x = x_ref[...][:N]
    aggp = aggp_ref[...]
    agg = aggp[:N] + aggp[NP:NP + N]
    deg = degp_ref[...]
    mean = agg * (1.0 / jnp.maximum(deg, 1.0))
    x3 = (jnp.dot(x, ws_ref[...], preferred_element_type=jnp.float32)
          + jnp.dot(mean, wn_ref[...], preferred_element_type=jnp.float32)
          + b_ref[...])
    batch = batch_ref[...]
    gid = jax.lax.broadcasted_iota(jnp.int32, (G, N), 0)
    onehot = (batch == gid).astype(jnp.float32)
    sums = jnp.dot(onehot, x3, preferred_element_type=jnp.float32)
    cnts = jnp.sum(onehot, axis=1, keepdims=True)
    h = sums * (1.0 / jnp.maximum(cnts, 1.0))
    h = jnp.maximum(
        jnp.dot(h, w1_ref[...], preferred_element_type=jnp.float32)
        + b1_ref[...], 0.0)
    out_ref[...] = (jnp.dot(h, w2_ref[...], preferred_element_type=jnp.float32)
                    + b2_ref[...])


def _final_layer(x, aggp, degp, ws, wn, b, batch, w1, b1, w2, b2):
    return pl.pallas_call(
        _final_body,
        out_shape=jax.ShapeDtypeStruct((G, C), jnp.float32),
    )(x, aggp, degp, ws, wn, b.reshape(1, H), batch.reshape(1, N),
      w1, b1.reshape(1, H), w2, b2.reshape(1, C))


def kernel(node_idx, edge_index, batch, emb, Wself0, Wneigh0, b0, Wself1,
           Wneigh1, b1, Wself2, Wneigh2, b2, gn_gamma0, gn_beta0, gn_alpha0,
           gn_gamma1, gn_beta1, gn_alpha1, mlp_W1, mlp_b1, mlp_W2, mlp_b2):
    node_idx = node_idx.astype(jnp.int32)
    src = edge_index[0].astype(jnp.int32)
    dst = edge_index[1].astype(jnp.int32)
    batch = batch.astype(jnp.int32)

    idx_pad = jnp.concatenate(
        [node_idx, jnp.zeros((NP - N,), jnp.int32)])

    x0p = _emb_gather(idx_pad, emb)

    zeros_nd = jnp.zeros((NP, D), jnp.float32)
    ones_kd = jnp.ones((K, D), jnp.float32)
    degp2 = _deg_scatter(dst, zeros_nd, ones_kd)
    degp = degp2[:N, :1] + degp2[NP:NP + N, :1]
    aggp0 = _agg_plain(x0p, src, dst, zeros_nd)
    aggp1_fn = lambda x: _agg_plain(x, src, dst, zeros_nd)

    x1 = _dense_layer(x0p, aggp0, degp, Wself0, Wneigh0, b0,
                      gn_gamma0, gn_beta0, gn_alpha0)
    aggp1 = aggp1_fn(x1)
    x2 = _dense_layer(x1, aggp1, degp, Wself1, Wneigh1, b1,
                      gn_gamma1, gn_beta1, gn_alpha1)
    aggp2 = aggp1_fn(x2)
    return _final_layer(x2, aggp2, degp, Wself2, Wneigh2, b2,
                        batch, mlp_W1, mlp_b1, mlp_W2, mlp_b2)


# speedup vs baseline: 9.6792x; 1.1443x over previous
import functools

import jax
import jax.numpy as jnp
from jax import lax
from jax.experimental import pallas as pl
from jax.experimental.pallas import tpu as pltpu
from jax.experimental.pallas import tpu_sc as plsc

N = 10000
E = 320000
D = 128
H = 128
C = 10
G = 64

NP = 10240
K = 80
NTILES = 32
EPT = E // NTILES
RPT = NP // 16

_MESH = plsc.VectorSubcoreMesh(core_axis_name="c", subcore_axis_name="s")


@functools.partial(
    pl.kernel,
    out_type=jax.ShapeDtypeStruct((NP, D), jnp.float32),
    mesh=_MESH,
    scratch_types=[
        pltpu.VMEM((K,), jnp.int32),
        pltpu.VMEM((K, D), jnp.float32),
        pltpu.SemaphoreType.DMA,
    ],
)
def _emb_gather(idx_hbm, emb_hbm, out_hbm, idx_v, rows_v, sem):
    cid = lax.axis_index("c")
    sid = lax.axis_index("s")
    wid = sid * 2 + cid
    base0 = wid * (NP // NTILES)

    def body(j, carry):
        base = base0 + j * K
        pltpu.sync_copy(idx_hbm.at[pl.ds(base, K)], idx_v)
        pltpu.async_copy(emb_hbm.at[idx_v], rows_v, sem).wait()
        pltpu.sync_copy(rows_v, out_hbm.at[pl.ds(base, K)])
        return carry

    lax.fori_loop(0, NP // NTILES // K, body, 0)


NCHUNK = EPT // K


@functools.partial(
    pl.kernel,
    out_type=jax.ShapeDtypeStruct((2 * NP, D), jnp.float32),
    mesh=_MESH,
    scratch_types=[
        pltpu.VMEM((EPT,), jnp.int32),
        pltpu.VMEM((EPT,), jnp.int32),
        pltpu.VMEM((K,), jnp.int32),
        pltpu.VMEM((K,), jnp.int32),
        pltpu.VMEM((K, D), jnp.float32),
        pltpu.VMEM((K, D), jnp.float32),
        pltpu.VMEM_SHARED((NP, D), jnp.float32),
        pltpu.SemaphoreType.DMA,
        pltpu.SemaphoreType.DMA,
        pltpu.SemaphoreType.DMA,
        pltpu.SemaphoreType.DMA,
    ],
)
def _agg_plain(x_hbm, src_hbm, dst_hbm, z_hbm, aggp_hbm,
               src_all, dst_all, dst_v0, dst_v1, rows_v0, rows_v1, agg_sh,
               gsem0, gsem1, ssem0, ssem1):
    cid = lax.axis_index("c")
    sid = lax.axis_index("s")

    rbase = sid * RPT
    pltpu.sync_copy(z_hbm.at[pl.ds(rbase, RPT)],
                    agg_sh.at[pl.ds(rbase, RPT)])
    ebase = cid * (E // 2) + sid * EPT
    pltpu.sync_copy(src_hbm.at[pl.ds(ebase, EPT)], src_all)
    pltpu.sync_copy(dst_hbm.at[pl.ds(ebase, EPT)], dst_all)
    plsc.subcore_barrier()

    dsts = (dst_v0, dst_v1)
    rows = (rows_v0, rows_v1)
    gsems = (gsem0, gsem1)
    ssems = (ssem0, ssem1)

    def copy_dst(j, b):
        for i in range(K // 16):
            dsts[b][pl.ds(i * 16, 16)] = dst_all[pl.ds(j * K + i * 16, 16)]

    def issue_gather(j, b):
        pltpu.async_copy(x_hbm.at[src_all.at[pl.ds(j * K, K)]], rows[b],
                         gsems[b])

    def wait_gather(b):
        pltpu.make_async_copy(x_hbm.at[src_all.at[pl.ds(0, K)]], rows[b],
                              gsems[b]).wait()

    def issue_scatter(b):
        pltpu.async_copy(rows[b], agg_sh.at[dsts[b]], ssems[b], add=True)

    def wait_scatter(b):
        pltpu.make_async_copy(rows[b], agg_sh.at[dsts[b]], ssems[b]).wait()

    copy_dst(0, 0)
    copy_dst(1, 1)
    issue_gather(0, 0)
    issue_gather(1, 1)

    def pair(m, carry):
        j = 2 * m
        wait_gather(0)
        issue_scatter(0)
        wait_scatter(0)
        copy_dst(j + 2, 0)
        issue_gather(j + 2, 0)
        wait_gather(1)
        issue_scatter(1)
        wait_scatter(1)
        nxt = jnp.minimum(j + 3, NCHUNK - 1)
        copy_dst(nxt, 1)
        issue_gather(nxt, 1)
        return carry

    lax.fori_loop(0, (NCHUNK - 1) // 2, pair, 0)
    wait_gather(0)
    issue_scatter(0)
    wait_scatter(0)
    wait_gather(1)
    plsc.subcore_barrier()

    obase = cid * NP + rbase
    pltpu.sync_copy(agg_sh.at[pl.ds(rbase, RPT)],
                    aggp_hbm.at[pl.ds(obase, RPT)])


@functools.partial(
    pl.kernel,
    out_type=jax.ShapeDtypeStruct((2 * NP, D), jnp.float32),
    mesh=_MESH,
    scratch_types=[
        pltpu.VMEM((K,), jnp.int32),
        pltpu.VMEM((K,), jnp.int32),
        pltpu.VMEM((K, D), jnp.float32),
        pltpu.VMEM_SHARED((NP, D), jnp.float32),
        pltpu.SemaphoreType.DMA,
        pltpu.SemaphoreType.DMA,
    ],
)
def _deg_scatter(dst_hbm, z_hbm, ones_hbm, degp_hbm, dst_v0, dst_v1,
                 ones_v, deg_sh, ssem0, ssem1):
    cid = lax.axis_index("c")
    sid = lax.axis_index("s")

    rbase = sid * RPT
    pltpu.sync_copy(z_hbm.at[pl.ds(rbase, RPT)],
                    deg_sh.at[pl.ds(rbase, RPT)])
    pltpu.sync_copy(ones_hbm, ones_v)
    plsc.subcore_barrier()

    ebase = cid * (E // 2) + sid * EPT
    dsts = (dst_v0, dst_v1)
    ssems = (ssem0, ssem1)

    def issue(b):
        pltpu.async_copy(ones_v, deg_sh.at[dsts[b]], ssems[b], add=True)

    def wait(b):
        pltpu.make_async_copy(ones_v, deg_sh.at[dsts[b]], ssems[b]).wait()

    pltpu.sync_copy(dst_hbm.at[pl.ds(ebase, K)], dst_v0)

    def pair(m, carry):
        j = 2 * m
        issue(0)
        pltpu.sync_copy(dst_hbm.at[pl.ds(ebase + (j + 1) * K, K)], dst_v1)
        wait(0)
        issue(1)
        pltpu.sync_copy(dst_hbm.at[pl.ds(ebase + (j + 2) * K, K)], dst_v0)
        wait(1)
        return carry

    lax.fori_loop(0, (EPT // K - 1) // 2, pair, 0)
    issue(0)
    wait(0)
    plsc.subcore_barrier()

    obase = cid * NP + rbase
    pltpu.sync_copy(deg_sh.at[pl.ds(rbase, RPT)],
                    degp_hbm.at[pl.ds(obase, RPT)])


def _dense_layer_body(x_ref, aggp_ref, degp_ref, ws_ref, wn_ref, b_ref,
                      gamma_ref, beta_ref, alpha_ref, out_ref):
    x = x_ref[...][:N]
    aggp = aggp_ref[...]
    agg = aggp[:N] + aggp[NP:NP + N]
    deg = degp_ref[...]
    mean = agg * (1.0 / jnp.maximum(deg, 1.0))
    y = (jnp.dot(x, ws_ref[...], preferred_element_type=jnp.float32)
         + jnp.dot(mean, wn_ref[...], preferred_element_type=jnp.float32)
         + b_ref[...])
    col_mean = jnp.mean(y, axis=0, keepdims=True)
    sub = y - alpha_ref[...] * col_mean
    var = jnp.mean(sub * sub, axis=0, keepdims=True)
    yn = gamma_ref[...] * sub * jax.lax.rsqrt(var + 1e-5) + beta_ref[...]
    out_ref[...] = jnp.maximum(yn, 0.0)


def _dense_layer(x, aggp, degp, ws, wn, b, gamma, beta, alpha):
    return pl.pallas_call(
        _dense_layer_body,
        out_shape=jax.ShapeDtypeStruct((N, H), jnp.float32),
    )(x, aggp, degp, ws, wn, b.reshape(1, H),
      gamma.reshape(1, H), beta.reshape(1, H), alpha.reshape(1, H))


def _final_body(x_ref, aggp_ref, degp_ref, ws_ref, wn_ref, b_ref,
                batch_ref, w1_ref, b1_ref, w2_ref, b2_ref, out_ref):
    x = x_ref[...][:N]
    aggp = aggp_ref[...]
    agg = aggp[:N] + aggp[NP:NP + N]
    deg = degp_ref[...]
    mean = agg * (1.0 / jnp.maximum(deg, 1.0))
    x3 = (jnp.dot(x, ws_ref[...], preferred_element_type=jnp.float32)
          + jnp.dot(mean, wn_ref[...], preferred_element_type=jnp.float32)
          + b_ref[...])
    batch = batch_ref[...]
    gid = jax.lax.broadcasted_iota(jnp.int32, (G, N), 0)
    onehot = (batch == gid).astype(jnp.float32)
    sums = jnp.dot(onehot, x3, preferred_element_type=jnp.float32)
    cnts = jnp.sum(onehot, axis=1, keepdims=True)
    h = sums * (1.0 / jnp.maximum(cnts, 1.0))
    h = jnp.maximum(
        jnp.dot(h, w1_ref[...], preferred_element_type=jnp.float32)
        + b1_ref[...], 0.0)
    out_ref[...] = (jnp.dot(h, w2_ref[...], preferred_element_type=jnp.float32)
                    + b2_ref[...])


def _final_layer(x, aggp, degp, ws, wn, b, batch, w1, b1, w2, b2):
    return pl.pallas_call(
        _final_body,
        out_shape=jax.ShapeDtypeStruct((G, C), jnp.float32),
    )(x, aggp, degp, ws, wn, b.reshape(1, H), batch.reshape(1, N),
      w1, b1.reshape(1, H), w2, b2.reshape(1, C))


def kernel(node_idx, edge_index, batch, emb, Wself0, Wneigh0, b0, Wself1,
           Wneigh1, b1, Wself2, Wneigh2, b2, gn_gamma0, gn_beta0, gn_alpha0,
           gn_gamma1, gn_beta1, gn_alpha1, mlp_W1, mlp_b1, mlp_W2, mlp_b2):
    node_idx = node_idx.astype(jnp.int32)
    src = edge_index[0].astype(jnp.int32)
    dst = edge_index[1].astype(jnp.int32)
    batch = batch.astype(jnp.int32)

    idx_pad = jnp.concatenate(
        [node_idx, jnp.zeros((NP - N,), jnp.int32)])

    x0p = _emb_gather(idx_pad, emb)

    zeros_nd = jnp.zeros((NP, D), jnp.float32)
    ones_kd = jnp.ones((K, D), jnp.float32)
    degp2 = _deg_scatter(dst, zeros_nd, ones_kd)
    degp = degp2[:N, :1] + degp2[NP:NP + N, :1]
    aggp0 = _agg_plain(x0p, src, dst, zeros_nd)
    aggp1_fn = lambda x: _agg_plain(x, src, dst, zeros_nd)

    x1 = _dense_layer(x0p, aggp0, degp, Wself0, Wneigh0, b0,
                      gn_gamma0, gn_beta0, gn_alpha0)
    aggp1 = aggp1_fn(x1)
    x2 = _dense_layer(x1, aggp1, degp, Wself1, Wneigh1, b1,
                      gn_gamma1, gn_beta1, gn_alpha1)
    aggp2 = aggp1_fn(x2)
    return _final_layer(x2, aggp2, degp, Wself2, Wneigh2, b2,
                        batch, mlp_W1, mlp_b1, mlp_W2, mlp_b2)


# speedup vs baseline: 10.6263x; 1.0979x over previous
import functools

import jax
import jax.numpy as jnp
from jax import lax
from jax.experimental import pallas as pl
from jax.experimental.pallas import tpu as pltpu
from jax.experimental.pallas import tpu_sc as plsc

N = 10000
E = 320000
D = 128
H = 128
C = 10
G = 64

NP = 10240
K = 80
NTILES = 32
EPT = E // NTILES
RPT = NP // 16

_MESH = plsc.VectorSubcoreMesh(core_axis_name="c", subcore_axis_name="s")


@functools.partial(
    pl.kernel,
    out_type=jax.ShapeDtypeStruct((NP, D), jnp.float32),
    mesh=_MESH,
    scratch_types=[
        pltpu.VMEM((K,), jnp.int32),
        pltpu.VMEM((K, D), jnp.float32),
        pltpu.SemaphoreType.DMA,
    ],
)
def _emb_gather(idx_hbm, emb_hbm, out_hbm, idx_v, rows_v, sem):
    cid = lax.axis_index("c")
    sid = lax.axis_index("s")
    wid = sid * 2 + cid
    base0 = wid * (NP // NTILES)

    def body(j, carry):
        base = base0 + j * K
        pltpu.sync_copy(idx_hbm.at[pl.ds(base, K)], idx_v)
        pltpu.async_copy(emb_hbm.at[idx_v], rows_v, sem).wait()
        pltpu.sync_copy(rows_v, out_hbm.at[pl.ds(base, K)])
        return carry

    lax.fori_loop(0, NP // NTILES // K, body, 0)


KC = 72
NMAIN = 138
KT = 64


@functools.partial(
    pl.kernel,
    out_type=jax.ShapeDtypeStruct((2 * NP, D), jnp.float32),
    mesh=_MESH,
    scratch_types=[
        pltpu.VMEM((EPT,), jnp.int32),
        pltpu.VMEM((EPT,), jnp.int32),
        pltpu.VMEM((KC,), jnp.int32),
        pltpu.VMEM((KC,), jnp.int32),
        pltpu.VMEM((KC,), jnp.int32),
        pltpu.VMEM((KT,), jnp.int32),
        pltpu.VMEM((KC, D), jnp.float32),
        pltpu.VMEM((KC, D), jnp.float32),
        pltpu.VMEM((KC, D), jnp.float32),
        pltpu.VMEM_SHARED((NP, D), jnp.float32),
        pltpu.SemaphoreType.DMA,
        pltpu.SemaphoreType.DMA,
        pltpu.SemaphoreType.DMA,
        pltpu.SemaphoreType.DMA,
        pltpu.SemaphoreType.DMA,
        pltpu.SemaphoreType.DMA,
    ],
)
def _agg_plain(x_hbm, src_hbm, dst_hbm, z_hbm, aggp_hbm,
               src_all, dst_all, dst_v0, dst_v1, dst_v2, dst_t,
               rows_v0, rows_v1, rows_v2, agg_sh,
               gsem0, gsem1, gsem2, ssem0, ssem1, ssem2):
    cid = lax.axis_index("c")
    sid = lax.axis_index("s")

    rbase = sid * RPT
    pltpu.sync_copy(z_hbm.at[pl.ds(rbase, RPT)],
                    agg_sh.at[pl.ds(rbase, RPT)])
    ebase = cid * (E // 2) + sid * EPT
    pltpu.sync_copy(src_hbm.at[pl.ds(ebase, EPT)], src_all)
    pltpu.sync_copy(dst_hbm.at[pl.ds(ebase, EPT)], dst_all)
    plsc.subcore_barrier()

    dsts = (dst_v0, dst_v1, dst_v2)
    rows = (rows_v0, rows_v1, rows_v2)
    gsems = (gsem0, gsem1, gsem2)
    ssems = (ssem0, ssem1, ssem2)

    def copy_dst(j, b):
        for o in (0, 16, 32, 48, 56):
            dsts[b][pl.ds(o, 16)] = dst_all[pl.ds(j * KC + o, 16)]

    def issue_gather(j, b):
        pltpu.async_copy(x_hbm.at[src_all.at[pl.ds(j * KC, KC)]], rows[b],
                         gsems[b])

    def wait_gather(b):
        pltpu.make_async_copy(x_hbm.at[src_all.at[pl.ds(0, KC)]], rows[b],
                              gsems[b]).wait()

    def issue_scatter(b):
        pltpu.async_copy(rows[b], agg_sh.at[dsts[b]], ssems[b], add=True)

    def wait_scatter(b):
        pltpu.make_async_copy(rows[b], agg_sh.at[dsts[b]], ssems[b]).wait()

    copy_dst(0, 0)
    copy_dst(1, 1)
    issue_gather(0, 0)
    issue_gather(1, 1)
    wait_gather(0)
    issue_scatter(0)
    copy_dst(2, 2)
    issue_gather(2, 2)
    wait_gather(1)
    issue_scatter(1)
    wait_scatter(0)
    copy_dst(3, 0)
    issue_gather(3, 0)

    def triple(m, carry):
        j0 = 3 * m + 2
        for p in range(3):
            j = j0 + p
            b = (2 + p) % 3
            bm1 = (1 + p) % 3
            wait_gather(b)
            issue_scatter(b)
            wait_scatter(bm1)
            copy_dst(j + 2, bm1)
            issue_gather(j + 2, bm1)
        return carry

    lax.fori_loop(0, 44, triple, 0)
    wait_gather(2)
    issue_scatter(2)
    wait_scatter(1)
    copy_dst(136, 1)
    issue_gather(136, 1)
    wait_gather(0)
    issue_scatter(0)
    wait_scatter(2)
    copy_dst(137, 2)
    issue_gather(137, 2)
    wait_gather(1)
    issue_scatter(1)
    wait_scatter(0)
    wait_gather(2)
    issue_scatter(2)
    wait_scatter(1)
    wait_scatter(2)
    for i in range(KT // 16):
        dst_t[pl.ds(i * 16, 16)] = dst_all[pl.ds(NMAIN * KC + i * 16, 16)]
    rows_tail = rows_v0.at[pl.ds(0, KT)]
    pltpu.async_copy(x_hbm.at[src_all.at[pl.ds(NMAIN * KC, KT)]],
                     rows_tail, gsem0)
    pltpu.make_async_copy(x_hbm.at[src_all.at[pl.ds(0, KT)]],
                          rows_tail, gsem0).wait()
    pltpu.async_copy(rows_tail, agg_sh.at[dst_t], ssem0, add=True)
    pltpu.make_async_copy(rows_tail, agg_sh.at[dst_t], ssem0).wait()
    plsc.subcore_barrier()

    obase = cid * NP + rbase
    pltpu.sync_copy(agg_sh.at[pl.ds(rbase, RPT)],
                    aggp_hbm.at[pl.ds(obase, RPT)])


@functools.partial(
    pl.kernel,
    out_type=jax.ShapeDtypeStruct((2 * NP, D), jnp.float32),
    mesh=_MESH,
    scratch_types=[
        pltpu.VMEM((K,), jnp.int32),
        pltpu.VMEM((K,), jnp.int32),
        pltpu.VMEM((K, D), jnp.float32),
        pltpu.VMEM_SHARED((NP, D), jnp.float32),
        pltpu.SemaphoreType.DMA,
        pltpu.SemaphoreType.DMA,
    ],
)
def _deg_scatter(dst_hbm, z_hbm, ones_hbm, degp_hbm, dst_v0, dst_v1,
                 ones_v, deg_sh, ssem0, ssem1):
    cid = lax.axis_index("c")
    sid = lax.axis_index("s")

    rbase = sid * RPT
    pltpu.sync_copy(z_hbm.at[pl.ds(rbase, RPT)],
                    deg_sh.at[pl.ds(rbase, RPT)])
    pltpu.sync_copy(ones_hbm, ones_v)
    plsc.subcore_barrier()

    ebase = cid * (E // 2) + sid * EPT
    dsts = (dst_v0, dst_v1)
    ssems = (ssem0, ssem1)

    def issue(b):
        pltpu.async_copy(ones_v, deg_sh.at[dsts[b]], ssems[b], add=True)

    def wait(b):
        pltpu.make_async_copy(ones_v, deg_sh.at[dsts[b]], ssems[b]).wait()

    pltpu.sync_copy(dst_hbm.at[pl.ds(ebase, K)], dst_v0)

    def pair(m, carry):
        j = 2 * m
        issue(0)
        pltpu.sync_copy(dst_hbm.at[pl.ds(ebase + (j + 1) * K, K)], dst_v1)
        wait(0)
        issue(1)
        pltpu.sync_copy(dst_hbm.at[pl.ds(ebase + (j + 2) * K, K)], dst_v0)
        wait(1)
        return carry

    lax.fori_loop(0, (EPT // K - 1) // 2, pair, 0)
    issue(0)
    wait(0)
    plsc.subcore_barrier()

    obase = cid * NP + rbase
    pltpu.sync_copy(deg_sh.at[pl.ds(rbase, RPT)],
                    degp_hbm.at[pl.ds(obase, RPT)])


def _dense_layer_body(x_ref, aggp_ref, degp_ref, ws_ref, wn_ref, b_ref,
                      gamma_ref, beta_ref, alpha_ref, out_ref):
    x = x_ref[...][:N]
    aggp = aggp_ref[...]
    agg = aggp[:N] + aggp[NP:NP + N]
    deg = degp_ref[...]
    mean = agg * (1.0 / jnp.maximum(deg, 1.0))
    y = (jnp.dot(x, ws_ref[...], preferred_element_type=jnp.float32)
         + jnp.dot(mean, wn_ref[...], preferred_element_type=jnp.float32)
         + b_ref[...])
    col_mean = jnp.mean(y, axis=0, keepdims=True)
    sub = y - alpha_ref[...] * col_mean
    var = jnp.mean(sub * sub, axis=0, keepdims=True)
    yn = gamma_ref[...] * sub * jax.lax.rsqrt(var + 1e-5) + beta_ref[...]
    out_ref[...] = jnp.maximum(yn, 0.0)


def _dense_layer(x, aggp, degp, ws, wn, b, gamma, beta, alpha):
    return pl.pallas_call(
        _dense_layer_body,
        out_shape=jax.ShapeDtypeStruct((N, H), jnp.float32),
    )(x, aggp, degp, ws, wn, b.reshape(1, H),
      gamma.reshape(1, H), beta.reshape(1, H), alpha.reshape(1, H))


def _final_body(x_ref, aggp_ref, degp_ref, ws_ref, wn_ref, b_ref,
                batch_ref, w1_ref, b1_ref, w2_ref, b2_ref, out_ref):
    x = x_ref[...][:N]
    aggp = aggp_ref[...]
    agg = aggp[:N] + aggp[NP:NP + N]
    deg = degp_ref[...]
    mean = agg * (1.0 / jnp.maximum(deg, 1.0))
    x3 = (jnp.dot(x, ws_ref[...], preferred_element_type=jnp.float32)
          + jnp.dot(mean, wn_ref[...], preferred_element_type=jnp.float32)
          + b_ref[...])
    batch = batch_ref[...]
    gid = jax.lax.broadcasted_iota(jnp.int32, (G, N), 0)
    onehot = (batch == gid).astype(jnp.float32)
    sums = jnp.dot(onehot, x3, preferred_element_type=jnp.float32)
    cnts = jnp.sum(onehot, axis=1, keepdims=True)
    h = sums * (1.0 / jnp.maximum(cnts, 1.0))
    h = jnp.maximum(
        jnp.dot(h, w1_ref[...], preferred_element_type=jnp.float32)
        + b1_ref[...], 0.0)
    out_ref[...] = (jnp.dot(h, w2_ref[...], preferred_element_type=jnp.float32)
                    + b2_ref[...])


def _final_layer(x, aggp, degp, ws, wn, b, batch, w1, b1, w2, b2):
    return pl.pallas_call(
        _final_body,
        out_shape=jax.ShapeDtypeStruct((G, C), jnp.float32),
    )(x, aggp, degp, ws, wn, b.reshape(1, H), batch.reshape(1, N),
      w1, b1.reshape(1, H), w2, b2.reshape(1, C))


def kernel(node_idx, edge_index, batch, emb, Wself0, Wneigh0, b0, Wself1,
           Wneigh1, b1, Wself2, Wneigh2, b2, gn_gamma0, gn_beta0, gn_alpha0,
           gn_gamma1, gn_beta1, gn_alpha1, mlp_W1, mlp_b1, mlp_W2, mlp_b2):
    node_idx = node_idx.astype(jnp.int32)
    src = edge_index[0].astype(jnp.int32)
    dst = edge_index[1].astype(jnp.int32)
    batch = batch.astype(jnp.int32)

    idx_pad = jnp.concatenate(
        [node_idx, jnp.zeros((NP - N,), jnp.int32)])

    x0p = _emb_gather(idx_pad, emb)

    zeros_nd = jnp.zeros((NP, D), jnp.float32)
    ones_kd = jnp.ones((K, D), jnp.float32)
    degp2 = _deg_scatter(dst, zeros_nd, ones_kd)
    degp = degp2[:N, :1] + degp2[NP:NP + N, :1]
    aggp0 = _agg_plain(x0p, src, dst, zeros_nd)
    aggp1_fn = lambda x: _agg_plain(x, src, dst, zeros_nd)

    x1 = _dense_layer(x0p, aggp0, degp, Wself0, Wneigh0, b0,
                      gn_gamma0, gn_beta0, gn_alpha0)
    aggp1 = aggp1_fn(x1)
    x2 = _dense_layer(x1, aggp1, degp, Wself1, Wneigh1, b1,
                      gn_gamma1, gn_beta1, gn_alpha1)
    aggp2 = aggp1_fn(x2)
    return _final_layer(x2, aggp2, degp, Wself2, Wneigh2, b2,
                        batch, mlp_W1, mlp_b1, mlp_W2, mlp_b2)
